# baseline (speedup 1.0000x reference)
import functools

import jax
import jax.numpy as jnp
from jax import lax
from jax.experimental import pallas as pl
from jax.experimental.pallas import tpu as pltpu
from jax.experimental.pallas import tpu_sc as plsc

B, L, D = 4096, 200, 64
R = B * L
NC, NS = 2, 16
NW = NC * NS
CHUNK = 128
ROWS_PER_W = R // NW
CHUNKS_PER_W = ROWS_PER_W // CHUNK


def _sc_gather(table, idx3):
    mesh = plsc.VectorSubcoreMesh(core_axis_name="c", subcore_axis_name="s")

    @functools.partial(
        pl.kernel,
        mesh=mesh,
        compiler_params=pltpu.CompilerParams(use_tc_tiling_on_sc=False),
        out_type=jax.ShapeDtypeStruct((R, D), jnp.float32),
        scratch_types=[
            pltpu.VMEM((CHUNKS_PER_W, CHUNK), jnp.int32),
            pltpu.VMEM((CHUNK, D), jnp.float32),
            pltpu.SemaphoreType.DMA,
        ],
    )
    def k(table_hbm, idx_hbm, out_hbm, idx_v, rows_v, gsem):
        wid = lax.axis_index("s") * NC + lax.axis_index("c")
        base = wid * ROWS_PER_W
        pltpu.sync_copy(idx_hbm.at[wid], idx_v)

        def body(j, carry):
            pltpu.async_copy(table_hbm.at[idx_v.at[j]], rows_v, gsem).wait()
            pltpu.sync_copy(rows_v, out_hbm.at[pl.ds(base + j * CHUNK, CHUNK)])
            return carry

        lax.fori_loop(0, CHUNKS_PER_W, body, 0, unroll=False)

    return k(table, idx3)


BLK = 2048


def _blend_body(g_ref, q_ref, w_ref, b_ref, o_ref):
    g = g_ref[...]
    q = q_ref[...]
    proj = jnp.dot(g, w_ref[...], preferred_element_type=jnp.float32)
    o_ref[...] = g + q * (proj + b_ref[...])


def _tc_blend(gathered, q2, wp, b2):
    return pl.pallas_call(
        _blend_body,
        grid=(R // BLK,),
        in_specs=[
            pl.BlockSpec((BLK, D), lambda i: (i, 0)),
            pl.BlockSpec((BLK, 1), lambda i: (i, 0)),
            pl.BlockSpec((D, D), lambda i: (0, 0)),
            pl.BlockSpec((1, D), lambda i: (0, 0)),
        ],
        out_specs=pl.BlockSpec((BLK, D), lambda i: (i, 0)),
        out_shape=jax.ShapeDtypeStruct((R, D), jnp.float32),
    )(gathered, q2, wp, b2)


def kernel(x, quantum_state, table, W, b):
    idx3 = x.reshape(NW, CHUNKS_PER_W, CHUNK)
    gathered = _sc_gather(table, idx3)
    wp = W - jnp.eye(D, dtype=W.dtype)
    out = _tc_blend(gathered, quantum_state.reshape(R, 1), wp, b.reshape(1, D))
    return out.reshape(B, L, D)


# speedup vs baseline: 1.2176x; 1.2176x over previous
import functools

import jax
import jax.numpy as jnp
from jax import lax
from jax.experimental import pallas as pl
from jax.experimental.pallas import tpu as pltpu
from jax.experimental.pallas import tpu_sc as plsc

B, L, D = 4096, 200, 64
R = B * L
NC, NS = 2, 16
NW = NC * NS
CHUNK = 128
ROWS_PER_W = R // NW
CHUNKS_PER_W = ROWS_PER_W // CHUNK
GC = 2
NSLAB = 4
GROUPS = CHUNKS_PER_W // GC
NCHUNKS = R // CHUNK


def _sc_gather(table, idx3):
    mesh = plsc.VectorSubcoreMesh(core_axis_name="c", subcore_axis_name="s")

    @functools.partial(
        pl.kernel,
        mesh=mesh,
        compiler_params=pltpu.CompilerParams(use_tc_tiling_on_sc=False),
        out_type=jax.ShapeDtypeStruct((NCHUNKS, CHUNK, D), jnp.float32),
        scratch_types=[
            pltpu.VMEM((CHUNKS_PER_W, CHUNK), jnp.int32),
            pltpu.VMEM((NSLAB, GC, CHUNK, D), jnp.float32),
            pltpu.SemaphoreType.DMA,
            pltpu.SemaphoreType.DMA,
            pltpu.SemaphoreType.DMA,
            pltpu.SemaphoreType.DMA,
            pltpu.SemaphoreType.DMA,
            pltpu.SemaphoreType.DMA,
            pltpu.SemaphoreType.DMA,
            pltpu.SemaphoreType.DMA,
        ],
    )
    def k(table_hbm, idx_hbm, out_hbm, idx_v, rows_v,
          g0, g1, g2, g3, w0, w1, w2, w3):
        gsem = (g0, g1, g2, g3)
        wsem = (w0, w1, w2, w3)
        wid = lax.axis_index("s") * NC + lax.axis_index("c")
        base_c = wid * CHUNKS_PER_W
        pltpu.sync_copy(idx_hbm.at[wid], idx_v)

        def fire_gathers(g, slab):
            for c in range(GC):
                pltpu.async_copy(
                    table_hbm.at[idx_v.at[g * GC + c]],
                    rows_v.at[slab, c],
                    gsem[slab],
                )

        def drain_gathers(slab):
            pltpu.make_async_copy(
                out_hbm.at[pl.ds(0, GC)], rows_v.at[slab], gsem[slab]
            ).wait()

        def fire_write(g, slab):
            pltpu.async_copy(
                rows_v.at[slab],
                out_hbm.at[pl.ds(base_c + g * GC, GC)],
                wsem[slab],
            )

        def drain_write(slab):
            pltpu.make_async_copy(
                rows_v.at[slab], out_hbm.at[pl.ds(0, GC)], wsem[slab]
            ).wait()

        fire_gathers(0, 0)
        fire_gathers(1, 1)

        def body(h, carry):
            for par in range(NSLAB):
                g = h * NSLAB + par
                slab2 = (par + 2) % NSLAB

                @pl.when(g >= 2)
                def _():
                    drain_write(slab2)

                @pl.when(g + 2 < GROUPS)
                def _():
                    fire_gathers(g + 2, slab2)

                drain_gathers(par)
                fire_write(g, par)
            return carry

        lax.fori_loop(0, GROUPS // NSLAB, body, 0, unroll=False)
        drain_write((GROUPS - 2) % NSLAB)
        drain_write((GROUPS - 1) % NSLAB)

    return k(table, idx3)


BLK2 = 2048


def _blend_body(g_ref, q_ref, w2_ref, b2_ref, o_ref):
    g = g_ref[...]
    q = q_ref[...]
    lane = lax.broadcasted_iota(jnp.int32, (BLK2, 128), 1)
    qq = jnp.where(lane < 64, q[:, 0:1], q[:, 1:2])
    proj = jnp.dot(g, w2_ref[...], preferred_element_type=jnp.float32)
    o_ref[...] = g + qq * (proj + b2_ref[...])


def _tc_blend(g2, q2, w2, b2):
    return pl.pallas_call(
        _blend_body,
        grid=(R // 2 // BLK2,),
        in_specs=[
            pl.BlockSpec((BLK2, 128), lambda i: (i, 0)),
            pl.BlockSpec((BLK2, 2), lambda i: (i, 0)),
            pl.BlockSpec((128, 128), lambda i: (0, 0)),
            pl.BlockSpec((1, 128), lambda i: (0, 0)),
        ],
        out_specs=pl.BlockSpec((BLK2, 128), lambda i: (i, 0)),
        out_shape=jax.ShapeDtypeStruct((R // 2, 128), jnp.float32),
    )(g2, q2, w2, b2)


def kernel(x, quantum_state, table, W, b):
    idx3 = x.reshape(NW, CHUNKS_PER_W, CHUNK)
    gathered = _sc_gather(table, idx3)
    g2 = gathered.reshape(R // 2, 128)
    wp = W - jnp.eye(D, dtype=W.dtype)
    zero = jnp.zeros((D, D), dtype=W.dtype)
    w2 = jnp.block([[wp, zero], [zero, wp]])
    b2 = jnp.concatenate([b, b]).reshape(1, 128)
    q2 = quantum_state.reshape(R // 2, 2)
    out = _tc_blend(g2, q2, w2, b2)
    return out.reshape(B, L, D)


# speedup vs baseline: 1.3806x; 1.1339x over previous
import functools

import jax
import jax.numpy as jnp
from jax import lax
from jax.experimental import pallas as pl
from jax.experimental.pallas import tpu as pltpu
from jax.experimental.pallas import tpu_sc as plsc

B, L, D = 4096, 200, 64
R = B * L
NC, NS = 2, 16
NW = NC * NS
CHUNK = 128
ROWS_PER_W = R // NW
CHUNKS_PER_W = ROWS_PER_W // CHUNK
GC = 2
NSLAB = 4
GROUPS = CHUNKS_PER_W // GC
NCHUNKS = R // CHUNK

W_BLK = 2048
NBLK = 489
PACKED_ROWS = NBLK * (W_BLK // 2)


def _linearize_body(t_ref, o_ref):
    t = t_ref[...]
    tr = jnp.transpose(t)
    o_ref[...] = jnp.concatenate([tr[: W_BLK // 2], tr[W_BLK // 2 :]], axis=1)


def _tc_linearize(tt):
    return pl.pallas_call(
        _linearize_body,
        grid=(NBLK,),
        in_specs=[pl.BlockSpec((64, W_BLK), lambda i: (0, i))],
        out_specs=pl.BlockSpec((W_BLK // 2, 128), lambda i: (i, 0)),
        out_shape=jax.ShapeDtypeStruct((PACKED_ROWS, 128), jnp.float32),
    )(tt)


def _sc_gather(table_lin, idx3):
    mesh = plsc.VectorSubcoreMesh(core_axis_name="c", subcore_axis_name="s")

    @functools.partial(
        pl.kernel,
        mesh=mesh,
        compiler_params=pltpu.CompilerParams(use_tc_tiling_on_sc=False),
        out_type=jax.ShapeDtypeStruct((NCHUNKS, CHUNK, D), jnp.float32),
        scratch_types=[
            pltpu.VMEM((CHUNKS_PER_W, CHUNK), jnp.int32),
            pltpu.VMEM((NSLAB, GC, CHUNK, D), jnp.float32),
            pltpu.SemaphoreType.DMA,
            pltpu.SemaphoreType.DMA,
            pltpu.SemaphoreType.DMA,
            pltpu.SemaphoreType.DMA,
            pltpu.SemaphoreType.DMA,
            pltpu.SemaphoreType.DMA,
            pltpu.SemaphoreType.DMA,
            pltpu.SemaphoreType.DMA,
        ],
    )
    def k(table_hbm, idx_hbm, out_hbm, idx_v, rows_v,
          g0, g1, g2, g3, w0, w1, w2, w3):
        gsem = (g0, g1, g2, g3)
        wsem = (w0, w1, w2, w3)
        wid = lax.axis_index("s") * NC + lax.axis_index("c")
        base_c = wid * CHUNKS_PER_W
        pltpu.sync_copy(idx_hbm.at[wid], idx_v)

        def fire_gathers(g, slab):
            for c in range(GC):
                pltpu.async_copy(
                    table_hbm.at[idx_v.at[g * GC + c]],
                    rows_v.at[slab, c],
                    gsem[slab],
                )

        def drain_gathers(slab):
            pltpu.make_async_copy(
                out_hbm.at[pl.ds(0, GC)], rows_v.at[slab], gsem[slab]
            ).wait()

        def fire_write(g, slab):
            pltpu.async_copy(
                rows_v.at[slab],
                out_hbm.at[pl.ds(base_c + g * GC, GC)],
                wsem[slab],
            )

        def drain_write(slab):
            pltpu.make_async_copy(
                rows_v.at[slab], out_hbm.at[pl.ds(0, GC)], wsem[slab]
            ).wait()

        fire_gathers(0, 0)
        fire_gathers(1, 1)

        def body(h, carry):
            for par in range(NSLAB):
                g = h * NSLAB + par
                slab2 = (par + 2) % NSLAB

                @pl.when(g >= 2)
                def _():
                    drain_write(slab2)

                @pl.when(g + 2 < GROUPS)
                def _():
                    fire_gathers(g + 2, slab2)

                drain_gathers(par)
                fire_write(g, par)
            return carry

        lax.fori_loop(0, GROUPS // NSLAB, body, 0, unroll=False)
        drain_write((GROUPS - 2) % NSLAB)
        drain_write((GROUPS - 1) % NSLAB)

    return k(table_lin, idx3)


BLK2 = 2048


def _blend_body(g_ref, qa_ref, qb_ref, w2_ref, b2_ref, o_ref):
    g = g_ref[...]
    qa = qa_ref[...]
    qb = qb_ref[...]
    lane = lax.broadcasted_iota(jnp.int32, (BLK2, 128), 1)
    qae = jax.lax.broadcast_in_dim(qa, (BLK2, 128), (0,))
    qbe = jax.lax.broadcast_in_dim(qb, (BLK2, 128), (0,))
    qq = jnp.where(lane < 64, qae, qbe)
    proj = jnp.dot(g, w2_ref[...], preferred_element_type=jnp.float32)
    o_ref[...] = g + qq * (proj + b2_ref[...])


def _tc_blend(g2, qa, qb, w2, b2):
    return pl.pallas_call(
        _blend_body,
        grid=(R // 2 // BLK2,),
        in_specs=[
            pl.BlockSpec((BLK2, 128), lambda i: (i, 0)),
            pl.BlockSpec((BLK2,), lambda i: (i,)),
            pl.BlockSpec((BLK2,), lambda i: (i,)),
            pl.BlockSpec((128, 128), lambda i: (0, 0)),
            pl.BlockSpec((1, 128), lambda i: (0, 0)),
        ],
        out_specs=pl.BlockSpec((BLK2, 128), lambda i: (i, 0)),
        out_shape=jax.ShapeDtypeStruct((R // 2, 128), jnp.float32),
    )(g2, qa, qb, w2, b2)


def kernel(x, quantum_state, table, W, b):
    xr = x.reshape(NW, CHUNKS_PER_W, CHUNK)
    idx3 = ((xr >> 11) << 11) + 2 * (xr & 1023) + ((xr >> 10) & 1)

    table_lin = _tc_linearize(table.T).reshape(2 * PACKED_ROWS, 64)
    gathered = _sc_gather(table_lin, idx3)
    g2 = gathered.reshape(R // 2, 128)

    wp = W - jnp.eye(D, dtype=W.dtype)
    zero = jnp.zeros((D, D), dtype=W.dtype)
    w2 = jnp.block([[wp, zero], [zero, wp]])
    b2 = jnp.concatenate([b, b]).reshape(1, 128)

    qflat = quantum_state.reshape(R)
    qa = qflat[0::2]
    qb = qflat[1::2]

    out = _tc_blend(g2, qa, qb, w2, b2)
    return out.reshape(B, L, D)


# speedup vs baseline: 1.7760x; 1.2864x over previous
import functools

import jax
import jax.numpy as jnp
from jax import lax
from jax.experimental import pallas as pl
from jax.experimental.pallas import tpu as pltpu
from jax.experimental.pallas import tpu_sc as plsc

B, L, D = 4096, 200, 64
R = B * L
NC, NS = 2, 16
NW = NC * NS
CHUNK = 128
ROWS_PER_W = R // NW
CHUNKS_PER_W = ROWS_PER_W // CHUNK
NSLAB = 4

W_BLK = 2048
NBLK = 489
PACKED_ROWS = NBLK * (W_BLK // 2)


def _linearize_body(t_ref, o_ref):
    t = t_ref[...]
    tr = jnp.transpose(t)
    o_ref[...] = jnp.concatenate([tr[: W_BLK // 2], tr[W_BLK // 2 :]], axis=1)


def _tc_linearize(tt):
    return pl.pallas_call(
        _linearize_body,
        grid=(NBLK,),
        in_specs=[pl.BlockSpec((64, W_BLK), lambda i: (0, i))],
        out_specs=pl.BlockSpec((W_BLK // 2, 128), lambda i: (i, 0)),
        out_shape=jax.ShapeDtypeStruct((PACKED_ROWS, 128), jnp.float32),
    )(tt)


def _sc_gather(table_lin, idx3):
    mesh = plsc.VectorSubcoreMesh(core_axis_name="c", subcore_axis_name="s")

    @functools.partial(
        pl.kernel,
        mesh=mesh,
        compiler_params=pltpu.CompilerParams(use_tc_tiling_on_sc=False),
        out_type=jax.ShapeDtypeStruct((R // 2, 128), jnp.float32),
        scratch_types=[
            pltpu.VMEM((CHUNKS_PER_W, CHUNK), jnp.int32),
            pltpu.VMEM((NSLAB, CHUNK, D), jnp.float32),
            pltpu.SemaphoreType.DMA,
            pltpu.SemaphoreType.DMA,
            pltpu.SemaphoreType.DMA,
            pltpu.SemaphoreType.DMA,
            pltpu.SemaphoreType.DMA,
            pltpu.SemaphoreType.DMA,
            pltpu.SemaphoreType.DMA,
            pltpu.SemaphoreType.DMA,
        ],
    )
    def k(table_hbm, idx_hbm, out_hbm, idx_v, rows_v,
          g0, g1, g2, g3, w0, w1, w2, w3):
        gsem = (g0, g1, g2, g3)
        wsem = (w0, w1, w2, w3)
        wid = lax.axis_index("s") * NC + lax.axis_index("c")
        base_c = wid * CHUNKS_PER_W
        pltpu.sync_copy(idx_hbm.at[wid], idx_v)

        def dst(g):
            cg = base_c + g
            prow = ((cg >> 5) << 11) + ((cg & 15) << 7)
            half = (cg >> 4) & 1
            return out_hbm.at[pl.ds(prow, CHUNK), pl.ds(half * D, D)]

        def fire_gather(g, slab):
            pltpu.async_copy(
                table_hbm.at[idx_v.at[g]], rows_v.at[slab], gsem[slab]
            )

        def drain_gather(slab):
            pltpu.make_async_copy(
                table_hbm.at[pl.ds(0, CHUNK)], rows_v.at[slab], gsem[slab]
            ).wait()

        def fire_write(g, slab):
            pltpu.async_copy(rows_v.at[slab], dst(g), wsem[slab])

        def drain_write(slab):
            pltpu.make_async_copy(rows_v.at[slab], dst(0), wsem[slab]).wait()

        fire_gather(0, 0)
        fire_gather(1, 1)

        def body(h, carry):
            for par in range(NSLAB):
                g = h * NSLAB + par
                slab2 = (par + 2) % NSLAB

                @pl.when(g >= 2)
                def _():
                    drain_write(slab2)

                @pl.when(g + 2 < CHUNKS_PER_W)
                def _():
                    fire_gather(g + 2, slab2)

                drain_gather(par)
                fire_write(g, par)
            return carry

        lax.fori_loop(0, CHUNKS_PER_W // NSLAB, body, 0, unroll=False)
        drain_write((CHUNKS_PER_W - 2) % NSLAB)
        drain_write((CHUNKS_PER_W - 1) % NSLAB)

    return k(table_lin, idx3)


BLK2 = 2048


def _blend_body(g_ref, qa_ref, qb_ref, w2_ref, b2_ref, o_ref):
    g = g_ref[...]
    qa = qa_ref[...]
    qb = qb_ref[...]
    lane = lax.broadcasted_iota(jnp.int32, (BLK2, 128), 1)
    qae = jax.lax.broadcast_in_dim(qa, (BLK2, 128), (0,))
    qbe = jax.lax.broadcast_in_dim(qb, (BLK2, 128), (0,))
    qq = jnp.where(lane < 64, qae, qbe)
    proj = jnp.dot(g, w2_ref[...], preferred_element_type=jnp.float32)
    o = g + qq * (proj + b2_ref[...])
    o_ref[...] = jnp.concatenate([o[:, :D], o[:, D:]], axis=0)


def _tc_blend(g2, qflat, w2, b2):
    return pl.pallas_call(
        _blend_body,
        grid=(R // 2 // BLK2,),
        in_specs=[
            pl.BlockSpec((BLK2, 128), lambda i: (i, 0)),
            pl.BlockSpec((BLK2,), lambda i: (2 * i,)),
            pl.BlockSpec((BLK2,), lambda i: (2 * i + 1,)),
            pl.BlockSpec((128, 128), lambda i: (0, 0)),
            pl.BlockSpec((1, 128), lambda i: (0, 0)),
        ],
        out_specs=pl.BlockSpec((2 * BLK2, D), lambda i: (i, 0)),
        out_shape=jax.ShapeDtypeStruct((R, D), jnp.float32),
    )(g2, qflat, qflat, w2, b2)


def kernel(x, quantum_state, table, W, b):
    xr = x.reshape(NW, CHUNKS_PER_W, CHUNK)
    idx3 = ((xr >> 11) << 11) + 2 * (xr & 1023) + ((xr >> 10) & 1)

    table_lin = _tc_linearize(table.T).reshape(2 * PACKED_ROWS, 64)
    g2 = _sc_gather(table_lin, idx3)

    wp = W - jnp.eye(D, dtype=W.dtype)
    zero = jnp.zeros((D, D), dtype=W.dtype)
    w2 = jnp.block([[wp, zero], [zero, wp]])
    b2 = jnp.concatenate([b, b]).reshape(1, 128)

    out = _tc_blend(g2, quantum_state.reshape(R), w2, b2)
    return out.reshape(B, L, D)


# speedup vs baseline: 1.9934x; 1.1224x over previous
import functools

import jax
import jax.numpy as jnp
from jax import lax
from jax.experimental import pallas as pl
from jax.experimental.pallas import tpu as pltpu
from jax.experimental.pallas import tpu_sc as plsc

B, L, D = 4096, 200, 64
R = B * L
NC, NS = 2, 16
NW = NC * NS
CHUNK = 128
ROWS_PER_W = R // NW
CHUNKS_PER_W = ROWS_PER_W // CHUNK
NSLAB = 4

W_BLK = 4096
NBLK = 245
PACKED_ROWS = NBLK * (W_BLK // 2)


def _linearize_body(t_ref, o_ref):
    t = t_ref[...]
    tr = jnp.transpose(t)
    o_ref[...] = jnp.concatenate([tr[: W_BLK // 2], tr[W_BLK // 2 :]], axis=1)


def _tc_linearize(tt):
    return pl.pallas_call(
        _linearize_body,
        grid=(NBLK,),
        in_specs=[pl.BlockSpec((64, W_BLK), lambda i: (0, i))],
        out_specs=pl.BlockSpec((W_BLK // 2, 128), lambda i: (i, 0)),
        out_shape=jax.ShapeDtypeStruct((PACKED_ROWS, 128), jnp.float32),
    )(tt)


def _sc_gather(table_lin, idx3):
    mesh = plsc.VectorSubcoreMesh(core_axis_name="c", subcore_axis_name="s")

    @functools.partial(
        pl.kernel,
        mesh=mesh,
        compiler_params=pltpu.CompilerParams(use_tc_tiling_on_sc=False),
        out_type=jax.ShapeDtypeStruct((R // 2, 128), jnp.float32),
        scratch_types=[
            pltpu.VMEM((CHUNKS_PER_W, CHUNK), jnp.int32),
            pltpu.VMEM((NSLAB, CHUNK, D), jnp.float32),
            pltpu.SemaphoreType.DMA,
            pltpu.SemaphoreType.DMA,
            pltpu.SemaphoreType.DMA,
            pltpu.SemaphoreType.DMA,
            pltpu.SemaphoreType.DMA,
            pltpu.SemaphoreType.DMA,
            pltpu.SemaphoreType.DMA,
            pltpu.SemaphoreType.DMA,
        ],
    )
    def k(table_hbm, idx_hbm, out_hbm, idx_v, rows_v,
          g0, g1, g2, g3, w0, w1, w2, w3):
        gsem = (g0, g1, g2, g3)
        wsem = (w0, w1, w2, w3)
        wid = lax.axis_index("s") * NC + lax.axis_index("c")
        base_c = wid * CHUNKS_PER_W
        pltpu.sync_copy(idx_hbm.at[wid], idx_v)

        def dst(g):
            cg = base_c + g
            prow = ((cg >> 5) << 11) + ((cg & 15) << 7)
            half = (cg >> 4) & 1
            return out_hbm.at[pl.ds(prow, CHUNK), pl.ds(half * D, D)]

        def fire_gather(g, slab):
            pltpu.async_copy(
                table_hbm.at[idx_v.at[g]], rows_v.at[slab], gsem[slab]
            )

        def drain_gather(slab):
            pltpu.make_async_copy(
                table_hbm.at[pl.ds(0, CHUNK)], rows_v.at[slab], gsem[slab]
            ).wait()

        def fire_write(g, slab):
            pltpu.async_copy(rows_v.at[slab], dst(g), wsem[slab])

        def drain_write(slab):
            pltpu.make_async_copy(rows_v.at[slab], dst(0), wsem[slab]).wait()

        fire_gather(0, 0)
        fire_gather(1, 1)

        def body(h, carry):
            for par in range(NSLAB):
                g = h * NSLAB + par
                slab2 = (par + 2) % NSLAB

                @pl.when(g >= 2)
                def _():
                    drain_write(slab2)

                @pl.when(g + 2 < CHUNKS_PER_W)
                def _():
                    fire_gather(g + 2, slab2)

                drain_gather(par)
                fire_write(g, par)
            return carry

        lax.fori_loop(0, CHUNKS_PER_W // NSLAB, body, 0, unroll=False)
        drain_write((CHUNKS_PER_W - 2) % NSLAB)
        drain_write((CHUNKS_PER_W - 1) % NSLAB)

    return k(table_lin, idx3)


BLK2 = 2048


def _blend_body(g_ref, qa_ref, qb_ref, w2_ref, b2_ref, o_ref):
    g = g_ref[...]
    qa = qa_ref[...]
    qb = qb_ref[...]
    lane = lax.broadcasted_iota(jnp.int32, (BLK2, 128), 1)
    qae = jax.lax.broadcast_in_dim(qa, (BLK2, 128), (0,))
    qbe = jax.lax.broadcast_in_dim(qb, (BLK2, 128), (0,))
    qq = jnp.where(lane < 64, qae, qbe)
    proj = jnp.dot(g, w2_ref[...], preferred_element_type=jnp.float32)
    o = g + qq * (proj + b2_ref[...])
    o_ref[...] = jnp.concatenate([o[:, :D], o[:, D:]], axis=0)


def _tc_blend(g2, qflat, w2, b2):
    return pl.pallas_call(
        _blend_body,
        grid=(R // 2 // BLK2,),
        in_specs=[
            pl.BlockSpec((BLK2, 128), lambda i: (i, 0)),
            pl.BlockSpec((BLK2,), lambda i: (2 * i,)),
            pl.BlockSpec((BLK2,), lambda i: (2 * i + 1,)),
            pl.BlockSpec((128, 128), lambda i: (0, 0)),
            pl.BlockSpec((1, 128), lambda i: (0, 0)),
        ],
        out_specs=pl.BlockSpec((2 * BLK2, D), lambda i: (i, 0)),
        out_shape=jax.ShapeDtypeStruct((R, D), jnp.float32),
    )(g2, qflat, qflat, w2, b2)


def kernel(x, quantum_state, table, W, b):
    xr = x.reshape(NW, CHUNKS_PER_W, CHUNK)
    idx3 = ((xr >> 12) << 12) + 2 * (xr & 2047) + ((xr >> 11) & 1)

    table_lin = _tc_linearize(table.T).reshape(2 * PACKED_ROWS, 64)
    g2 = _sc_gather(table_lin, idx3)

    wp = W - jnp.eye(D, dtype=W.dtype)
    zero = jnp.zeros((D, D), dtype=W.dtype)
    w2 = jnp.block([[wp, zero], [zero, wp]])
    b2 = jnp.concatenate([b, b]).reshape(1, 128)

    out = _tc_blend(g2, quantum_state.reshape(R), w2, b2)
    return out.reshape(B, L, D)


# speedup vs baseline: 2.1414x; 1.0742x over previous
import functools

import jax
import jax.numpy as jnp
from jax import lax
from jax.experimental import pallas as pl
from jax.experimental.pallas import tpu as pltpu
from jax.experimental.pallas import tpu_sc as plsc

B, L, D = 4096, 200, 64
R = B * L
NC, NS = 2, 16
NW = NC * NS
CHUNK = 128
ROWS_PER_W = R // NW
CHUNKS_PER_W = ROWS_PER_W // CHUNK
NSLAB = 4

W_BLK = 8192
NBLK = 123
PACKED_ROWS = NBLK * (W_BLK // 2)


def _linearize_body(t_ref, o_ref):
    t = t_ref[...]
    tr = jnp.transpose(t)
    o_ref[...] = jnp.concatenate([tr[: W_BLK // 2], tr[W_BLK // 2 :]], axis=1)


def _tc_linearize(tt):
    return pl.pallas_call(
        _linearize_body,
        grid=(NBLK,),
        in_specs=[pl.BlockSpec((64, W_BLK), lambda i: (0, i))],
        out_specs=pl.BlockSpec((W_BLK // 2, 128), lambda i: (i, 0)),
        out_shape=jax.ShapeDtypeStruct((PACKED_ROWS, 128), jnp.float32),
    )(tt)


def _sc_gather(table_lin, idx3):
    mesh = plsc.VectorSubcoreMesh(core_axis_name="c", subcore_axis_name="s")

    @functools.partial(
        pl.kernel,
        mesh=mesh,
        compiler_params=pltpu.CompilerParams(use_tc_tiling_on_sc=False),
        out_type=jax.ShapeDtypeStruct((R // 2, 128), jnp.float32),
        scratch_types=[
            pltpu.VMEM((CHUNKS_PER_W, CHUNK), jnp.int32),
            pltpu.VMEM((NSLAB, CHUNK, D), jnp.float32),
            pltpu.SemaphoreType.DMA,
            pltpu.SemaphoreType.DMA,
            pltpu.SemaphoreType.DMA,
            pltpu.SemaphoreType.DMA,
            pltpu.SemaphoreType.DMA,
            pltpu.SemaphoreType.DMA,
            pltpu.SemaphoreType.DMA,
            pltpu.SemaphoreType.DMA,
        ],
    )
    def k(table_hbm, idx_hbm, out_hbm, idx_v, rows_v,
          g0, g1, g2, g3, w0, w1, w2, w3):
        gsem = (g0, g1, g2, g3)
        wsem = (w0, w1, w2, w3)
        wid = lax.axis_index("s") * NC + lax.axis_index("c")
        base_c = wid * CHUNKS_PER_W
        pltpu.sync_copy(idx_hbm.at[wid], idx_v)

        def dst(g):
            cg = base_c + g
            prow = ((cg >> 5) << 11) + ((cg & 15) << 7)
            half = (cg >> 4) & 1
            return out_hbm.at[pl.ds(prow, CHUNK), pl.ds(half * D, D)]

        def fire_gather(g, slab):
            pltpu.async_copy(
                table_hbm.at[idx_v.at[g]], rows_v.at[slab], gsem[slab]
            )

        def drain_gather(slab):
            pltpu.make_async_copy(
                table_hbm.at[pl.ds(0, CHUNK)], rows_v.at[slab], gsem[slab]
            ).wait()

        def fire_write(g, slab):
            pltpu.async_copy(rows_v.at[slab], dst(g), wsem[slab])

        def drain_write(slab):
            pltpu.make_async_copy(rows_v.at[slab], dst(0), wsem[slab]).wait()

        fire_gather(0, 0)
        fire_gather(1, 1)

        def body(h, carry):
            for par in range(NSLAB):
                g = h * NSLAB + par
                slab2 = (par + 2) % NSLAB

                @pl.when(g >= 2)
                def _():
                    drain_write(slab2)

                @pl.when(g + 2 < CHUNKS_PER_W)
                def _():
                    fire_gather(g + 2, slab2)

                drain_gather(par)
                fire_write(g, par)
            return carry

        lax.fori_loop(0, CHUNKS_PER_W // NSLAB, body, 0, unroll=False)
        drain_write((CHUNKS_PER_W - 2) % NSLAB)
        drain_write((CHUNKS_PER_W - 1) % NSLAB)

    return k(table_lin, idx3)


BLK2 = 2048


def _blend_body(g_ref, qa_ref, qb_ref, w2_ref, b2_ref, o_ref):
    g = g_ref[...]
    qa = qa_ref[...]
    qb = qb_ref[...]
    lane = lax.broadcasted_iota(jnp.int32, (BLK2, 128), 1)
    qae = jax.lax.broadcast_in_dim(qa, (BLK2, 128), (0,))
    qbe = jax.lax.broadcast_in_dim(qb, (BLK2, 128), (0,))
    qq = jnp.where(lane < 64, qae, qbe)
    proj = jnp.dot(g, w2_ref[...], preferred_element_type=jnp.float32)
    o = g + qq * (proj + b2_ref[...])
    o_ref[...] = jnp.concatenate([o[:, :D], o[:, D:]], axis=0)


def _tc_blend(g2, qflat, w2, b2):
    return pl.pallas_call(
        _blend_body,
        grid=(R // 2 // BLK2,),
        in_specs=[
            pl.BlockSpec((BLK2, 128), lambda i: (i, 0)),
            pl.BlockSpec((BLK2,), lambda i: (2 * i,)),
            pl.BlockSpec((BLK2,), lambda i: (2 * i + 1,)),
            pl.BlockSpec((128, 128), lambda i: (0, 0)),
            pl.BlockSpec((1, 128), lambda i: (0, 0)),
        ],
        out_specs=pl.BlockSpec((2 * BLK2, D), lambda i: (i, 0)),
        out_shape=jax.ShapeDtypeStruct((R, D), jnp.float32),
    )(g2, qflat, qflat, w2, b2)


def kernel(x, quantum_state, table, W, b):
    xr = x.reshape(NW, CHUNKS_PER_W, CHUNK)
    idx3 = ((xr >> 13) << 13) + 2 * (xr & 4095) + ((xr >> 12) & 1)

    table_lin = _tc_linearize(table.T).reshape(2 * PACKED_ROWS, 64)
    g2 = _sc_gather(table_lin, idx3)

    wp = W - jnp.eye(D, dtype=W.dtype)
    zero = jnp.zeros((D, D), dtype=W.dtype)
    w2 = jnp.block([[wp, zero], [zero, wp]])
    b2 = jnp.concatenate([b, b]).reshape(1, 128)

    out = _tc_blend(g2, quantum_state.reshape(R), w2, b2)
    return out.reshape(B, L, D)
